# baseline (speedup 1.0000x reference)
import jax
import jax.numpy as jnp
from jax import lax
from jax.experimental import pallas as pl
from jax.experimental.pallas import tpu as pltpu

_T = 84
_EPS = 1e-5
_F32 = jnp.float32
_BF16 = jnp.bfloat16


def _dot(a, b):
    return jnp.dot(a, b, preferred_element_type=_F32)


def _dotr(a, b):
    return lax.dot_general(a, b, (((1,), (1,)), ((), ())),
                           preferred_element_type=_F32)


def _tree_sum(terms):
    while len(terms) > 1:
        nxt = [terms[i] + terms[i + 1] for i in range(0, len(terms) - 1, 2)]
        if len(terms) % 2:
            nxt.append(terms[-1])
        terms = nxt
    return terms[0]


_NR = 8
_TOP = 24
_STR = 112
_W = (_NR - 1) * _STR + 84
_LTOT = _TOP + _W + 20


def _dw_relu_st(xs, wdt_ref, bdt_ref, *, K, dil):
    C = xs.shape[1]
    pad = (K - 1) // 2 * dil
    wdt = wdt_ref[...]
    terms = [jnp.broadcast_to(bdt_ref[...], (_W, C))]
    for k in range(K):
        o = _TOP + k * dil - pad
        terms.append(wdt[k:k + 1, :] * xs[o:o + _W, :])
    return jnp.maximum(_tree_sum(terms), 0.0)


def _stack_pad(rows_):
    C = rows_[0].shape[1]
    gap = jnp.zeros((28, C), _F32)
    parts = [jnp.zeros((_TOP, C), _F32)]
    for r, x in enumerate(rows_):
        parts.append(x)
        parts.append(gap if r < _NR - 1 else jnp.zeros((20, C), _F32))
    return jnp.concatenate(parts, axis=0)


def _repad(z):
    C = z.shape[1]
    return jnp.concatenate(
        [jnp.zeros((_TOP, C), _F32), z, jnp.zeros((20, C), _F32)], axis=0)


def _split3(r):
    h0 = r.astype(_BF16).astype(_F32)
    r1 = r - h0
    h1 = r1.astype(_BF16).astype(_F32)
    return jnp.concatenate([h0, h1, r1 - h1], axis=0)


def _ln_pair(y, gt_ref, bt_ref, tsr_ref, scat_ref, s0t_ref, s1t_ref):
    C = y.shape[1]
    ys = [y[r * _STR:r * _STR + 84, :] for r in range(_NR)]
    t = lax.broadcasted_iota(jnp.int32, (84, C), 0)
    first = t < tsr_ref[...]
    inv_c = _F32(1.0 / C)
    scat = scat_ref[...]
    s0t = s0t_ref[...]
    s1t = s1t_ref[...]
    rsum = lambda m, i: jnp.sum(m[3 * i:3 * i + 3, :], axis=0, keepdims=True)

    def stat_rows(r):
        a0 = jnp.sum(jnp.where(first, r, 0.0), axis=0, keepdims=True)
        a1 = jnp.sum(r, axis=0, keepdims=True) - a0
        return jnp.concatenate([_split3(a0), _split3(a1)], axis=1)

    st = _dot(jnp.concatenate([stat_rows(yr) for yr in ys], axis=0), scat)
    cmu = [rsum(st, r) * inv_c for r in range(_NR)]
    cpair = jnp.concatenate([_split3(c) for c in cmu], axis=0)
    X0 = _dot(cpair, s0t)
    X1 = _dot(cpair, s1t)

    def var_rows(r, i):
        d0 = jnp.where(first, r - rsum(X0, i), 0.0)
        d1 = jnp.where(first, 0.0, r - rsum(X1, i))
        q0 = jnp.sum(d0 * d0, axis=0, keepdims=True)
        q1 = jnp.sum(d1 * d1, axis=0, keepdims=True)
        return jnp.concatenate([_split3(q0), _split3(q1)], axis=1)

    sv = _dot(jnp.concatenate([var_rows(yr, r) for r, yr in enumerate(ys)],
                              axis=0), scat)
    cvar = [rsum(sv, r) * inv_c for r in range(_NR)]
    bq = lambda v: v.astype(_BF16).astype(_F32)
    mv = jnp.concatenate(
        [bq(v) for r in range(_NR) for v in (cmu[r], cvar[r])], axis=0)
    Q0 = _dot(mv, s0t)
    Q1 = _dot(mv, s1t)
    g = gt_ref[...]
    b = bt_ref[...]
    gap = jnp.zeros((28, C), _F32)
    parts = []
    for r in range(_NR):
        mu_g = jnp.where(first, Q0[2 * r:2 * r + 1, :], Q1[2 * r:2 * r + 1, :])
        var_g = jnp.where(first, Q0[2 * r + 1:2 * r + 2, :],
                          Q1[2 * r + 1:2 * r + 2, :])
        parts.append((ys[r] - mu_g) * lax.rsqrt(var_g + _EPS) * g + b)
        if r < _NR - 1:
            parts.append(gap)
    return jnp.concatenate(parts, axis=0)


def _pair_forward(feat_ref, pv_ref,
                  wd1, bd1, wp1, bq1, g1, bb1, ts1, sc1, s0t1, s1t1,
                  wd2, bd2, wp2, bq2, g2, bb2, ts2, sc2, s0t2, s1t2,
                  wd3, bd3, wp3, bq3, g3, bb3, ts3, sc3, s0t3, s1t3,
                  wdp, bdp, wpp, bqp, gp, bbp, tsp, scp, s0tp, s1tp,
                  w1a, w1b, fb1, fw2, fb2, fw3, fb3):
    xs = _stack_pad([jnp.transpose(feat_ref[r]) for r in range(_NR)])
    h = _dw_relu_st(xs, wd1, bd1, K=11, dil=1)
    y1 = jnp.maximum(_dot(h, wp1[...]) + bq1[...], 0.0)
    z1 = _ln_pair(y1, g1, bb1, ts1, sc1, s0t1, s1t1)
    h = _dw_relu_st(_repad(z1), wd2, bd2, K=11, dil=2)
    y2 = jnp.maximum(_dot(h, wp2[...]) + bq2[...], 0.0)
    z2 = _ln_pair(y2, g2, bb2, ts2, sc2, s0t2, s1t2)
    h = _dw_relu_st(_repad(z2), wd3, bd3, K=11, dil=4)
    y3 = jnp.maximum(_dot(h, wp3[...]) + bq3[...], 0.0)
    z3 = _ln_pair(y3, g3, bb3, ts3, sc3, s0t3, s1t3)

    xsp = _stack_pad([pv_ref[r] for r in range(_NR)])
    hp = _dw_relu_st(xsp, wdp, bdp, K=11, dil=1)
    yp = jnp.maximum(_dot(hp, wpp[...]) + bqp[...], 0.0)
    zp = _ln_pair(yp, gp, bbp, tsp, scp, s0tp, s1tp)

    mrow = lambda z: jnp.concatenate(
        [jnp.mean(z[r * _STR:r * _STR + 84, :], axis=0, keepdims=True)
         for r in range(_NR)], axis=0)
    p3 = mrow(z3)
    pp = mrow(zp)
    h1 = _dotr(p3, w1a[...]) + _dotr(pp, w1b[...]) + fb1[...]
    h2 = _dotr(h1, fw2[...]) + fb2[...]
    prod = (h2.astype(_BF16).astype(_F32)
            * fw3[...].astype(_BF16).astype(_F32))
    return jnp.sum(prod, axis=1, keepdims=True) + fb3[...]


def _fused_kernel(feat_ref, pv_ref, *args):
    wargs, o_ref = args[:-1], args[-1]
    o_ref[...] = _pair_forward(feat_ref, pv_ref, *wargs)


def _w2d(shape):
    n = len(shape)
    return pl.BlockSpec(tuple(shape), lambda i, n=n: (0,) * n)


def _ln_pack(g_grid, b_grid, s0, s1, tstar):
    C = g_grid.shape[0]
    return [g_grid.T, b_grid.T, tstar.reshape(1, C),
            jnp.concatenate([s0, s1], axis=0), s0.T, s1.T]


def kernel(feat, pv,
           w1d, b1d, w1p, b1p,
           w2d, b2d, w2p, b2p,
           w3d, b3d, w3p, b3p,
           wpd, bpd, wpp, bpp,
           ln1_g_grid, ln1_b_grid, ln1_s0, ln1_s1, ln1_tstar,
           ln2_g_grid, ln2_b_grid, ln2_s0, ln2_s1, ln2_tstar,
           ln3_g_grid, ln3_b_grid, ln3_s0, ln3_s1, ln3_tstar,
           lnp_g_grid, lnp_b_grid, lnp_s0, lnp_s1, lnp_tstar,
           fc_w1a, fc_w1b, fc_b1, fc_w1s, fc_b1s,
           fc_w2, fc_b2, fc_w3, fc_b3):
    B = feat.shape[0]
    pvr = pv.reshape(B, _T, 16 * 54)

    row = lambda v: v.reshape(1, -1)
    operands = [
        feat, pvr,
        w1d.T, row(b1d), w1p.T, row(b1p),
        *_ln_pack(ln1_g_grid, ln1_b_grid, ln1_s0, ln1_s1, ln1_tstar),
        w2d.T, row(b2d), w2p.T, row(b2p),
        *_ln_pack(ln2_g_grid, ln2_b_grid, ln2_s0, ln2_s1, ln2_tstar),
        w3d.T, row(b3d), w3p.T, row(b3p),
        *_ln_pack(ln3_g_grid, ln3_b_grid, ln3_s0, ln3_s1, ln3_tstar),
        wpd.T, row(bpd), wpp.T, row(bpp),
        *_ln_pack(lnp_g_grid, lnp_b_grid, lnp_s0, lnp_s1, lnp_tstar),
        fc_w1a, fc_w1b, row(fc_b1), fc_w2, row(fc_b2), fc_w3, fc_b3,
    ]
    in_specs = (
        [pl.BlockSpec((_NR, 769, _T), lambda i: (i, 0, 0)),
         pl.BlockSpec((_NR, _T, 864), lambda i: (i, 0, 0))]
        + [_w2d(op.shape) for op in operands[2:]]
    )
    out = pl.pallas_call(
        _fused_kernel,
        out_shape=jax.ShapeDtypeStruct((B // _NR, _NR, 1), _F32),
        grid=(B // _NR,),
        in_specs=in_specs,
        out_specs=pl.BlockSpec((None, _NR, 1), lambda i: (i, 0, 0)),
        compiler_params=pltpu.CompilerParams(
            dimension_semantics=("parallel",)),
    )(*operands)
    return out.reshape(B)


# speedup vs baseline: 1.0392x; 1.0392x over previous
import jax
import jax.numpy as jnp
from jax import lax
from jax.experimental import pallas as pl
from jax.experimental.pallas import tpu as pltpu

_T = 84
_EPS = 1e-5
_F32 = jnp.float32
_BF16 = jnp.bfloat16


def _dot(a, b):
    return jnp.dot(a, b, preferred_element_type=_F32)


def _dotr(a, b):
    return lax.dot_general(a, b, (((1,), (1,)), ((), ())),
                           preferred_element_type=_F32)


def _tree_sum(terms):
    while len(terms) > 1:
        nxt = [terms[i] + terms[i + 1] for i in range(0, len(terms) - 1, 2)]
        if len(terms) % 2:
            nxt.append(terms[-1])
        terms = nxt
    return terms[0]


_NR = 8
_TOP = 24
_STR = 112
_W = (_NR - 1) * _STR + 84
_LTOT = _TOP + _W + 20


def _dw_relu_st(xs, wdt_ref, bdt_ref, *, K, dil):
    C = xs.shape[1]
    pad = (K - 1) // 2 * dil
    wdt = wdt_ref[...]
    terms = [jnp.broadcast_to(bdt_ref[...], (_W, C))]
    for k in range(K):
        o = _TOP + k * dil - pad
        terms.append(wdt[k:k + 1, :] * xs[o:o + _W, :])
    return jnp.maximum(_tree_sum(terms), 0.0)


def _stack_pad(rows_):
    C = rows_[0].shape[1]
    gap = jnp.zeros((28, C), _F32)
    parts = [jnp.zeros((_TOP, C), _F32)]
    for r, x in enumerate(rows_):
        parts.append(x)
        parts.append(gap if r < _NR - 1 else jnp.zeros((20, C), _F32))
    return jnp.concatenate(parts, axis=0)


def _repad(z):
    C = z.shape[1]
    return jnp.concatenate(
        [jnp.zeros((_TOP, C), _F32), z, jnp.zeros((20, C), _F32)], axis=0)


def _split3(r):
    h0 = r.astype(_BF16).astype(_F32)
    r1 = r - h0
    h1 = r1.astype(_BF16).astype(_F32)
    return jnp.concatenate([h0, h1, r1 - h1], axis=0)


def _ln_pair(y, gt_ref, bt_ref, tsr_ref, scat_ref, s0t_ref, s1t_ref):
    C = y.shape[1]
    ys = [y[r * _STR:r * _STR + 84, :] for r in range(_NR)]
    t = lax.broadcasted_iota(jnp.int32, (84, C), 0)
    first = t < tsr_ref[...]
    inv_c = _F32(1.0 / C)
    scat = scat_ref[...]
    s0t = s0t_ref[...]
    s1t = s1t_ref[...]
    rsum = lambda m, i: jnp.sum(m[3 * i:3 * i + 3, :], axis=0, keepdims=True)

    def stat_rows(r):
        a0 = jnp.sum(jnp.where(first, r, 0.0), axis=0, keepdims=True)
        a1 = jnp.sum(r, axis=0, keepdims=True) - a0
        return jnp.concatenate([_split3(a0), _split3(a1)], axis=1)

    st = _dot(jnp.concatenate([stat_rows(yr) for yr in ys], axis=0), scat)
    cmu = [rsum(st, r) * inv_c for r in range(_NR)]
    cpair = jnp.concatenate([_split3(c) for c in cmu], axis=0)
    X0 = _dot(cpair, s0t)
    X1 = _dot(cpair, s1t)

    def var_rows(r, i):
        d0 = jnp.where(first, r - rsum(X0, i), 0.0)
        d1 = jnp.where(first, 0.0, r - rsum(X1, i))
        q0 = jnp.sum(d0 * d0, axis=0, keepdims=True)
        q1 = jnp.sum(d1 * d1, axis=0, keepdims=True)
        return jnp.concatenate([_split3(q0), _split3(q1)], axis=1)

    sv = _dot(jnp.concatenate([var_rows(yr, r) for r, yr in enumerate(ys)],
                              axis=0), scat)
    cvar = [rsum(sv, r) * inv_c for r in range(_NR)]
    bq = lambda v: v.astype(_BF16).astype(_F32)
    mv = jnp.concatenate(
        [bq(v) for r in range(_NR) for v in (cmu[r], cvar[r])], axis=0)
    Q0 = _dot(mv, s0t)
    Q1 = _dot(mv, s1t)
    g = gt_ref[...]
    b = bt_ref[...]
    gap = jnp.zeros((28, C), _F32)
    parts = []
    for r in range(_NR):
        mu_g = jnp.where(first, Q0[2 * r:2 * r + 1, :], Q1[2 * r:2 * r + 1, :])
        var_g = jnp.where(first, Q0[2 * r + 1:2 * r + 2, :],
                          Q1[2 * r + 1:2 * r + 2, :])
        parts.append((ys[r] - mu_g) * lax.rsqrt(var_g + _EPS) * g + b)
        if r < _NR - 1:
            parts.append(gap)
    return jnp.concatenate(parts, axis=0)


def _pair_forward(feat_ref, pv_ref,
                  wd1, bd1, wp1, bq1, g1, bb1, ts1, sc1, s0t1, s1t1,
                  wd2, bd2, wp2, bq2, g2, bb2, ts2, sc2, s0t2, s1t2,
                  wd3, bd3, wp3, bq3, g3, bb3, ts3, sc3, s0t3, s1t3,
                  wdp, bdp, wpp, bqp, gp, bbp, tsp, scp, s0tp, s1tp,
                  w1a, w1b, fb1, fw2, fb2, fw3, fb3):
    xs = _stack_pad([feat_ref[r] for r in range(_NR)])
    h = _dw_relu_st(xs, wd1, bd1, K=11, dil=1)
    y1 = jnp.maximum(_dot(h, wp1[...]) + bq1[...], 0.0)
    z1 = _ln_pair(y1, g1, bb1, ts1, sc1, s0t1, s1t1)
    h = _dw_relu_st(_repad(z1), wd2, bd2, K=11, dil=2)
    y2 = jnp.maximum(_dot(h, wp2[...]) + bq2[...], 0.0)
    z2 = _ln_pair(y2, g2, bb2, ts2, sc2, s0t2, s1t2)
    h = _dw_relu_st(_repad(z2), wd3, bd3, K=11, dil=4)
    y3 = jnp.maximum(_dot(h, wp3[...]) + bq3[...], 0.0)
    z3 = _ln_pair(y3, g3, bb3, ts3, sc3, s0t3, s1t3)

    xsp = _stack_pad([pv_ref[r] for r in range(_NR)])
    hp = _dw_relu_st(xsp, wdp, bdp, K=11, dil=1)
    yp = jnp.maximum(_dot(hp, wpp[...]) + bqp[...], 0.0)
    zp = _ln_pair(yp, gp, bbp, tsp, scp, s0tp, s1tp)

    mrow = lambda z: jnp.concatenate(
        [jnp.mean(z[r * _STR:r * _STR + 84, :], axis=0, keepdims=True)
         for r in range(_NR)], axis=0)
    p3 = mrow(z3)
    pp = mrow(zp)
    h1 = _dotr(p3, w1a[...]) + _dotr(pp, w1b[...]) + fb1[...]
    h2 = _dotr(h1, fw2[...]) + fb2[...]
    prod = (h2.astype(_BF16).astype(_F32)
            * fw3[...].astype(_BF16).astype(_F32))
    return jnp.sum(prod, axis=1, keepdims=True) + fb3[...]


def _fused_kernel(feat_ref, pv_ref, *args):
    wargs, o_ref = args[:-1], args[-1]
    o_ref[...] = _pair_forward(feat_ref, pv_ref, *wargs)


def _w2d(shape):
    n = len(shape)
    return pl.BlockSpec(tuple(shape), lambda i, n=n: (0,) * n)


def _ln_pack(g_grid, b_grid, s0, s1, tstar):
    C = g_grid.shape[0]
    return [g_grid.T, b_grid.T, tstar.reshape(1, C),
            jnp.concatenate([s0, s1], axis=0), s0.T, s1.T]


def kernel(feat, pv,
           w1d, b1d, w1p, b1p,
           w2d, b2d, w2p, b2p,
           w3d, b3d, w3p, b3p,
           wpd, bpd, wpp, bpp,
           ln1_g_grid, ln1_b_grid, ln1_s0, ln1_s1, ln1_tstar,
           ln2_g_grid, ln2_b_grid, ln2_s0, ln2_s1, ln2_tstar,
           ln3_g_grid, ln3_b_grid, ln3_s0, ln3_s1, ln3_tstar,
           lnp_g_grid, lnp_b_grid, lnp_s0, lnp_s1, lnp_tstar,
           fc_w1a, fc_w1b, fc_b1, fc_w1s, fc_b1s,
           fc_w2, fc_b2, fc_w3, fc_b3):
    B = feat.shape[0]
    feat_t = jnp.swapaxes(feat, 1, 2)
    pvr = pv.reshape(B, _T, 16 * 54)

    row = lambda v: v.reshape(1, -1)
    operands = [
        feat_t, pvr,
        w1d.T, row(b1d), w1p.T, row(b1p),
        *_ln_pack(ln1_g_grid, ln1_b_grid, ln1_s0, ln1_s1, ln1_tstar),
        w2d.T, row(b2d), w2p.T, row(b2p),
        *_ln_pack(ln2_g_grid, ln2_b_grid, ln2_s0, ln2_s1, ln2_tstar),
        w3d.T, row(b3d), w3p.T, row(b3p),
        *_ln_pack(ln3_g_grid, ln3_b_grid, ln3_s0, ln3_s1, ln3_tstar),
        wpd.T, row(bpd), wpp.T, row(bpp),
        *_ln_pack(lnp_g_grid, lnp_b_grid, lnp_s0, lnp_s1, lnp_tstar),
        fc_w1a, fc_w1b, row(fc_b1), fc_w2, row(fc_b2), fc_w3, fc_b3,
    ]
    in_specs = (
        [pl.BlockSpec((_NR, _T, 769), lambda i: (i, 0, 0)),
         pl.BlockSpec((_NR, _T, 864), lambda i: (i, 0, 0))]
        + [_w2d(op.shape) for op in operands[2:]]
    )
    out = pl.pallas_call(
        _fused_kernel,
        out_shape=jax.ShapeDtypeStruct((B // _NR, _NR, 1), _F32),
        grid=(B // _NR,),
        in_specs=in_specs,
        out_specs=pl.BlockSpec((None, _NR, 1), lambda i: (i, 0, 0)),
        compiler_params=pltpu.CompilerParams(
            dimension_semantics=("parallel",)),
    )(*operands)
    return out.reshape(B)
